# Initial kernel scaffold; baseline (speedup 1.0000x reference)
import functools

import jax
import jax.numpy as jnp
from jax import lax
from jax.experimental import pallas as pl
from jax.experimental.pallas import tpu as pltpu
from jax.experimental.pallas import tpu_sc as plsc

N = 10000
E = 320000
D = 128

NC = 2
NS = 16
NW = NC * NS
EPT = E // NW
ROWS_PT = N // NS
CH = 128
NFULL = EPT // CH
REM = EPT - NFULL * CH
G = CH // 16


def _edge_body(with_stats, *refs):
    if with_stats:
        (src_h, dst_h, ea_h, as_h, ad_h, h_hbm, we_h, ae_h,
         acc_p, den_p, m_out, cnt_p, sa_p,
         src_l, dst_l, ea_l, as_l, ad_l, den_l, we_l, ae_l,
         srcv, dstv, srcv16, dstv16, exb, rows, rows16, mbuf, acc_sh, sem,
         cnt_l, sa_l) = refs
    else:
        (src_h, dst_h, ea_h, as_h, ad_h, h_hbm, we_h, ae_h,
         acc_p, den_p, m_out,
         src_l, dst_l, ea_l, as_l, ad_l, den_l, we_l, ae_l,
         srcv, dstv, srcv16, dstv16, exb, rows, rows16, mbuf, acc_sh,
         sem) = refs
        cnt_p = sa_p = cnt_l = sa_l = None

    c_ax = lax.axis_index("c")
    s_ax = lax.axis_index("s")
    w = c_ax * NS + s_ax
    base_e = w * EPT

    pltpu.sync_copy(src_h.at[pl.ds(base_e, EPT)], src_l)
    pltpu.sync_copy(dst_h.at[pl.ds(base_e, EPT)], dst_l)
    pltpu.sync_copy(ea_h.at[pl.ds(base_e, EPT)], ea_l)
    pltpu.sync_copy(as_h, as_l)
    pltpu.sync_copy(ad_h, ad_l)
    pltpu.sync_copy(we_h, we_l)
    pltpu.sync_copy(ae_h, ae_l)

    cacc = jnp.zeros((16,), jnp.float32)
    for i in range(D // 16):
        cacc = cacc + we_l[pl.ds(i * 16, 16)] * ae_l[pl.ds(i * 16, 16)]
    c = jnp.sum(cacc)

    def mx_body(i, carry):
        ma, md = carry
        ma = jnp.maximum(ma, as_l[pl.ds(i * 16, 16)])
        md = jnp.maximum(md, ad_l[pl.ds(i * 16, 16)])
        return ma, md

    neg = jnp.full((16,), -1e30, jnp.float32)
    ma, md = lax.fori_loop(0, N // 16, mx_body, (neg, neg))
    m_ub = jnp.max(ma) + jnp.max(md) + jnp.maximum(c, 0.0)
    m = jnp.maximum(m_ub, 0.2 * m_ub)

    zv = jnp.zeros((16,), jnp.float32)

    def z_body(i, _):
        den_l[pl.ds(i * 16, 16)] = zv
        if with_stats:
            cnt_l[pl.ds(i * 16, 16)] = zv
            sa_l[pl.ds(i * 16, 16)] = zv
        return 0

    lax.fori_loop(0, N // 16, z_body, 0)

    def zr_body(i, _):
        for j in range(G):
            rows[i, pl.ds(j * 16, 16)] = zv
        return 0

    lax.fori_loop(0, CH, zr_body, 0)
    for k in range(5):
        pltpu.sync_copy(rows.at[pl.ds(0, 125)],
                        acc_sh.at[pl.ds(s_ax * ROWS_PT + k * 125, 125)])
    plsc.subcore_barrier()

    ones16 = jnp.ones((16,), jnp.float32)

    def chunk_body(k, _):
        eb = k * CH
        for j in range(G):
            srcv[pl.ds(j * 16, 16)] = src_l[pl.ds(eb + j * 16, 16)]
            dstv[pl.ds(j * 16, 16)] = dst_l[pl.ds(eb + j * 16, 16)]
        cp = pltpu.async_copy(h_hbm.at[srcv], rows, sem)
        for j in range(G):
            sv = srcv[pl.ds(j * 16, 16)]
            dv = dstv[pl.ds(j * 16, 16)]
            eag = ea_l[pl.ds(eb + j * 16, 16)]
            lg = (plsc.load_gather(as_l, [sv]) + plsc.load_gather(ad_l, [dv])
                  + c * eag)
            lg = jnp.maximum(lg, 0.2 * lg)
            exv = jnp.exp(lg - m)
            exb[pl.ds(j * 16, 16)] = exv
            plsc.addupdate_scatter(den_l, [dv], exv)
            if with_stats:
                plsc.addupdate_scatter(cnt_l, [dv], ones16)
                plsc.addupdate_scatter(sa_l, [dv], eag)
        cp.wait()

        def sc_body(i, _):
            e = exb[i]
            for j in range(G):
                rows[i, pl.ds(j * 16, 16)] = rows[i, pl.ds(j * 16, 16)] * e
            return 0

        lax.fori_loop(0, CH, sc_body, 0)
        pltpu.sync_copy(rows, acc_sh.at[dstv], add=True)
        return 0

    lax.fori_loop(0, NFULL, chunk_body, 0)

    eb = NFULL * CH
    srcv16[...] = src_l[pl.ds(eb, 16)]
    dstv16[...] = dst_l[pl.ds(eb, 16)]
    cp = pltpu.async_copy(h_hbm.at[srcv16], rows16, sem)
    sv = srcv16[...]
    dv = dstv16[...]
    eag = ea_l[pl.ds(eb, 16)]
    lg = plsc.load_gather(as_l, [sv]) + plsc.load_gather(ad_l, [dv]) + c * eag
    lg = jnp.maximum(lg, 0.2 * lg)
    exv = jnp.exp(lg - m)
    exb[pl.ds(0, 16)] = exv
    plsc.addupdate_scatter(den_l, [dv], exv)
    if with_stats:
        plsc.addupdate_scatter(cnt_l, [dv], ones16)
        plsc.addupdate_scatter(sa_l, [dv], eag)
    cp.wait()

    def rem_body(i, _):
        e = exb[i]
        for j in range(G):
            rows16[i, pl.ds(j * 16, 16)] = rows16[i, pl.ds(j * 16, 16)] * e
        return 0

    lax.fori_loop(0, REM, rem_body, 0)
    pltpu.sync_copy(rows16, acc_sh.at[dstv16], add=True)

    plsc.subcore_barrier()

    pltpu.sync_copy(acc_sh.at[pl.ds(s_ax * ROWS_PT, ROWS_PT)],
                    acc_p.at[c_ax, pl.ds(s_ax * ROWS_PT, ROWS_PT)])
    pltpu.sync_copy(den_l, den_p.at[w])
    if with_stats:
        pltpu.sync_copy(cnt_l, cnt_p.at[w])
        pltpu.sync_copy(sa_l, sa_p.at[w])

    @pl.when(w == 0)
    def _():
        mbuf[...] = jnp.zeros((16,), jnp.float32) + m
        pltpu.sync_copy(mbuf, m_out)


def _make_edge_kernel(with_stats):
    mesh = plsc.VectorSubcoreMesh(core_axis_name="c", subcore_axis_name="s")
    out_type = [
        jax.ShapeDtypeStruct((NC, N, D), jnp.float32),
        jax.ShapeDtypeStruct((NW, N), jnp.float32),
        jax.ShapeDtypeStruct((16,), jnp.float32),
    ]
    scratch = [
        pltpu.VMEM((EPT,), jnp.int32),
        pltpu.VMEM((EPT,), jnp.int32),
        pltpu.VMEM((EPT,), jnp.float32),
        pltpu.VMEM((N,), jnp.float32),
        pltpu.VMEM((N,), jnp.float32),
        pltpu.VMEM((N,), jnp.float32),
        pltpu.VMEM((D,), jnp.float32),
        pltpu.VMEM((D,), jnp.float32),
        pltpu.VMEM((CH,), jnp.int32),
        pltpu.VMEM((CH,), jnp.int32),
        pltpu.VMEM((REM,), jnp.int32),
        pltpu.VMEM((REM,), jnp.int32),
        pltpu.VMEM((CH,), jnp.float32),
        pltpu.VMEM((CH, D), jnp.float32),
        pltpu.VMEM((REM, D), jnp.float32),
        pltpu.VMEM((16,), jnp.float32),
        pltpu.VMEM_SHARED((N, D), jnp.float32),
        pltpu.SemaphoreType.DMA,
    ]
    if with_stats:
        out_type += [
            jax.ShapeDtypeStruct((NW, N), jnp.float32),
            jax.ShapeDtypeStruct((NW, N), jnp.float32),
        ]
        scratch += [
            pltpu.VMEM((N,), jnp.float32),
            pltpu.VMEM((N,), jnp.float32),
        ]
    return pl.kernel(
        functools.partial(_edge_body, with_stats),
        mesh=mesh,
        out_type=out_type,
        scratch_types=scratch,
    )


_edge_kernel1 = _make_edge_kernel(True)
_edge_kernel2 = _make_edge_kernel(False)


def _k1_body(x_ref, wres_ref, bres_ref, w1_ref, asw_ref, adw_ref, alpha_ref,
             xresa_ref, h1t_ref, as1_ref, ad1_ref):
    x = x_ref[...]
    xr = jnp.dot(x, wres_ref[...], preferred_element_type=jnp.float32)
    xr = xr + bres_ref[...][None, :]
    h = jnp.dot(xr, w1_ref[...], preferred_element_type=jnp.float32)
    al = jnp.sum(alpha_ref[...])
    xresa_ref[...] = al * xr
    h1t_ref[...] = h
    as1_ref[...] = jnp.sum(h * asw_ref[...][None, :], axis=1)
    ad1_ref[...] = jnp.sum(h * adw_ref[...][None, :], axis=1)


_k1 = pl.pallas_call(
    _k1_body,
    out_shape=[
        jax.ShapeDtypeStruct((N, D), jnp.float32),
        jax.ShapeDtypeStruct((N, D), jnp.float32),
        jax.ShapeDtypeStruct((N,), jnp.float32),
        jax.ShapeDtypeStruct((N,), jnp.float32),
    ],
)


def _k3_body(accp_ref, denp_ref, cntp_ref, sap_ref, as_ref, ad_ref, h1t_ref,
             m_ref, we_ref, ae_ref, b1_ref, w2_ref, asw_ref, adw_ref,
             h2t_ref, as2_ref, ad2_ref, la_ref):
    cnt = jnp.sum(cntp_ref[...], axis=0)
    sa = jnp.sum(sap_ref[...], axis=0)
    la = sa / jnp.maximum(cnt, 1.0)
    c = jnp.sum(we_ref[...] * ae_ref[...])
    m = jnp.max(m_ref[...])
    sl = as_ref[...] + ad_ref[...] + c * la
    sl = jnp.maximum(sl, 0.2 * sl)
    exsl = jnp.exp(sl - m)
    den = jnp.sum(denp_ref[...], axis=0) + exsl
    acc = jnp.sum(accp_ref[...], axis=0) + exsl[:, None] * h1t_ref[...]
    out1 = acc / den[:, None] + b1_ref[...][None, :]
    h1 = jax.nn.elu(out1)
    h2 = jnp.dot(h1, w2_ref[...], preferred_element_type=jnp.float32)
    h2t_ref[...] = h2
    as2_ref[...] = jnp.sum(h2 * asw_ref[...][None, :], axis=1)
    ad2_ref[...] = jnp.sum(h2 * adw_ref[...][None, :], axis=1)
    la_ref[...] = la


_k3 = pl.pallas_call(
    _k3_body,
    out_shape=[
        jax.ShapeDtypeStruct((N, D), jnp.float32),
        jax.ShapeDtypeStruct((N,), jnp.float32),
        jax.ShapeDtypeStruct((N,), jnp.float32),
        jax.ShapeDtypeStruct((N,), jnp.float32),
    ],
)


def _k5_body(accp_ref, denp_ref, as_ref, ad_ref, la_ref, h2t_ref, m_ref,
             we_ref, ae_ref, b2_ref, xresa_ref, out_ref):
    c = jnp.sum(we_ref[...] * ae_ref[...])
    m = jnp.max(m_ref[...])
    sl = as_ref[...] + ad_ref[...] + c * la_ref[...]
    sl = jnp.maximum(sl, 0.2 * sl)
    exsl = jnp.exp(sl - m)
    den = jnp.sum(denp_ref[...], axis=0) + exsl
    acc = jnp.sum(accp_ref[...], axis=0) + exsl[:, None] * h2t_ref[...]
    out_ref[...] = acc / den[:, None] + b2_ref[...][None, :] + xresa_ref[...]


_k5 = pl.pallas_call(
    _k5_body,
    out_shape=jax.ShapeDtypeStruct((N, D), jnp.float32),
)


def kernel(x, edge_index, edge_attr, W_res, b_res, W1, att_src1, att_dst1,
           W_edge1, att_edge1, b1, W2, att_src2, att_dst2, W_edge2,
           att_edge2, b2, alpha):
    src0 = edge_index[0]
    dst0 = edge_index[1]
    ea = edge_attr[:, 0]
    alpha2d = jnp.reshape(alpha, (1, 1))
    we1 = jnp.reshape(W_edge1, (D,))
    we2 = jnp.reshape(W_edge2, (D,))

    xres_a, h1t, as1, ad1 = _k1(x, W_res, b_res, W1, att_src1, att_dst1,
                                alpha2d)
    acc1, den1, m1, cnt1, sa1 = _edge_kernel1(src0, dst0, ea, as1, ad1, h1t,
                                              we1, att_edge1)
    h2t, as2, ad2, la = _k3(acc1, den1, cnt1, sa1, as1, ad1, h1t, m1, we1,
                            att_edge1, b1, W2, att_src2, att_dst2)
    acc2, den2, m2 = _edge_kernel2(src0, dst0, ea, as2, ad2, h2t, we2,
                                   att_edge2)
    out = _k5(acc2, den2, as2, ad2, la, h2t, m2, we2, att_edge2, b2, xres_a)
    return out


# speedup vs baseline: 26.6756x; 26.6756x over previous
import functools

import jax
import jax.numpy as jnp
from jax import lax
from jax.experimental import pallas as pl
from jax.experimental.pallas import tpu as pltpu
from jax.experimental.pallas import tpu_sc as plsc

N = 10000
E = 320000
D = 128

NC = 2
NS = 16
NW = NC * NS
EPT = E // NW
ROWS_AL = 624
ROWS_TL = N - NS * ROWS_AL
CH = 128
NFULL = EPT // CH
REM = EPT - NFULL * CH
G = CH // 16


def _scale_rows(rows_ref, exb_ref, ngroups):
    def body(r, _):
        e16 = plsc.load_gather(exb_ref, [jnp.zeros((16,), jnp.int32) + r])
        for j in range(G):
            rows_ref[r, pl.ds(j * 16, 16)] = (
                rows_ref[r, pl.ds(j * 16, 16)] * e16)
        return 0

    lax.fori_loop(0, ngroups * 16, body, 0)


def _edge_body(with_stats, *refs):
    if with_stats:
        (src_h, dst_h, ea_h, as_h, ad_h, h_hbm, c_h, m_h,
         acc_p, den_p, cnt_p, sa_p,
         as_l, ad_l, c_l, m_l,
         srcv, dstv, eav, srcv16, dstv16, eav16, exb, ones_b, rows, rows16,
         den_sh, acc_sh, sem, cnt_sh, sa_sh) = refs
    else:
        (src_h, dst_h, ea_h, as_h, ad_h, h_hbm, c_h, m_h,
         acc_p, den_p,
         as_l, ad_l, c_l, m_l,
         srcv, dstv, eav, srcv16, dstv16, eav16, exb, ones_b, rows, rows16,
         den_sh, acc_sh, sem) = refs
        cnt_p = sa_p = cnt_sh = sa_sh = None

    c_ax = lax.axis_index("c")
    s_ax = lax.axis_index("s")
    w = c_ax * NS + s_ax
    base_e = w * EPT

    pltpu.sync_copy(as_h, as_l)
    pltpu.sync_copy(ad_h, ad_l)
    pltpu.sync_copy(c_h, c_l)
    pltpu.sync_copy(m_h, m_l)

    cv = c_l[...]
    mv = m_l[...]

    zv = jnp.zeros((16,), jnp.float32)
    ones16 = jnp.ones((16,), jnp.float32)
    for j in range(G):
        exb[pl.ds(j * 16, 16)] = zv
        ones_b[pl.ds(j * 16, 16)] = ones16

    def zr_body(i, _):
        for j in range(G):
            rows[i, pl.ds(j * 16, 16)] = zv
        return 0

    lax.fori_loop(0, CH, zr_body, 0)
    base_r = s_ax * ROWS_AL
    for k in range(4):
        pltpu.sync_copy(rows, acc_sh.at[pl.ds(base_r + k * CH, CH)])
        pltpu.sync_copy(exb, den_sh.at[pl.ds(base_r + k * CH, CH)])
        if with_stats:
            pltpu.sync_copy(exb, cnt_sh.at[pl.ds(base_r + k * CH, CH)])
            pltpu.sync_copy(exb, sa_sh.at[pl.ds(base_r + k * CH, CH)])
    tail_sz = ROWS_AL - 4 * CH
    pltpu.sync_copy(rows.at[pl.ds(0, tail_sz)],
                    acc_sh.at[pl.ds(base_r + 4 * CH, tail_sz)])
    pltpu.sync_copy(exb.at[pl.ds(0, tail_sz)],
                    den_sh.at[pl.ds(base_r + 4 * CH, tail_sz)])
    if with_stats:
        pltpu.sync_copy(exb.at[pl.ds(0, tail_sz)],
                        cnt_sh.at[pl.ds(base_r + 4 * CH, tail_sz)])
        pltpu.sync_copy(exb.at[pl.ds(0, tail_sz)],
                        sa_sh.at[pl.ds(base_r + 4 * CH, tail_sz)])

    @pl.when(s_ax == NS - 1)
    def _():
        pltpu.sync_copy(rows.at[pl.ds(0, ROWS_TL)],
                        acc_sh.at[pl.ds(NS * ROWS_AL, ROWS_TL)])
        pltpu.sync_copy(exb.at[pl.ds(0, ROWS_TL)],
                        den_sh.at[pl.ds(NS * ROWS_AL, ROWS_TL)])
        if with_stats:
            pltpu.sync_copy(exb.at[pl.ds(0, ROWS_TL)],
                            cnt_sh.at[pl.ds(NS * ROWS_AL, ROWS_TL)])
            pltpu.sync_copy(exb.at[pl.ds(0, ROWS_TL)],
                            sa_sh.at[pl.ds(NS * ROWS_AL, ROWS_TL)])

    plsc.subcore_barrier()

    def chunk_body(k, _):
        eb = base_e + k * CH
        pltpu.sync_copy(src_h.at[pl.ds(eb, CH)], srcv)
        pltpu.sync_copy(dst_h.at[pl.ds(eb, CH)], dstv)
        pltpu.sync_copy(ea_h.at[pl.ds(eb, CH)], eav)
        cp = pltpu.async_copy(h_hbm.at[srcv], rows, sem)
        for j in range(G):
            sv = srcv[pl.ds(j * 16, 16)]
            dv = dstv[pl.ds(j * 16, 16)]
            eag = eav[pl.ds(j * 16, 16)]
            lg = (plsc.load_gather(as_l, [sv]) + plsc.load_gather(ad_l, [dv])
                  + cv * eag)
            lg = jnp.maximum(lg, 0.2 * lg)
            exb[pl.ds(j * 16, 16)] = jnp.exp(lg - mv)
        pltpu.sync_copy(exb, den_sh.at[dstv], add=True)
        if with_stats:
            pltpu.sync_copy(ones_b, cnt_sh.at[dstv], add=True)
            pltpu.sync_copy(eav, sa_sh.at[dstv], add=True)
        cp.wait()
        _scale_rows(rows, exb, G)
        pltpu.sync_copy(rows, acc_sh.at[dstv], add=True)
        return 0

    lax.fori_loop(0, NFULL, chunk_body, 0)

    eb = base_e + NFULL * CH
    pltpu.sync_copy(src_h.at[pl.ds(eb, REM)], srcv16)
    pltpu.sync_copy(dst_h.at[pl.ds(eb, REM)], dstv16)
    pltpu.sync_copy(ea_h.at[pl.ds(eb, REM)], eav16)
    cp = pltpu.async_copy(h_hbm.at[srcv16], rows16, sem)
    sv = srcv16[...]
    dv = dstv16[...]
    eag = eav16[...]
    lg = plsc.load_gather(as_l, [sv]) + plsc.load_gather(ad_l, [dv]) + cv * eag
    lg = jnp.maximum(lg, 0.2 * lg)
    exb[pl.ds(0, 16)] = jnp.exp(lg - mv)
    pltpu.sync_copy(exb.at[pl.ds(0, REM)], den_sh.at[dstv16], add=True)
    if with_stats:
        pltpu.sync_copy(ones_b.at[pl.ds(0, REM)], cnt_sh.at[dstv16], add=True)
        pltpu.sync_copy(eav16, sa_sh.at[dstv16], add=True)
    cp.wait()
    _scale_rows(rows16, exb, 1)
    pltpu.sync_copy(rows16, acc_sh.at[dstv16], add=True)

    plsc.subcore_barrier()

    def cp1d(src_sh, dst_h, so, do, n):
        for o in range(0, n, CH):
            sz = min(CH, n - o)
            pltpu.sync_copy(src_sh.at[pl.ds(so + o, sz)],
                            exb.at[pl.ds(0, sz)])
            pltpu.sync_copy(exb.at[pl.ds(0, sz)],
                            dst_h.at[pl.ds(do + o, sz)])

    pltpu.sync_copy(acc_sh.at[pl.ds(base_r, ROWS_AL)],
                    acc_p.at[c_ax, pl.ds(base_r, ROWS_AL)])
    cp1d(den_sh, den_p, base_r, c_ax * N + base_r, ROWS_AL)
    if with_stats:
        cp1d(cnt_sh, cnt_p, base_r, c_ax * N + base_r, ROWS_AL)
        cp1d(sa_sh, sa_p, base_r, c_ax * N + base_r, ROWS_AL)

    @pl.when(s_ax == NS - 1)
    def _():
        pltpu.sync_copy(acc_sh.at[pl.ds(NS * ROWS_AL, ROWS_TL)],
                        acc_p.at[c_ax, pl.ds(NS * ROWS_AL, ROWS_TL)])
        cp1d(den_sh, den_p, NS * ROWS_AL, c_ax * N + NS * ROWS_AL, ROWS_TL)
        if with_stats:
            cp1d(cnt_sh, cnt_p, NS * ROWS_AL, c_ax * N + NS * ROWS_AL,
                 ROWS_TL)
            cp1d(sa_sh, sa_p, NS * ROWS_AL, c_ax * N + NS * ROWS_AL, ROWS_TL)


def _make_edge_kernel(with_stats):
    mesh = plsc.VectorSubcoreMesh(core_axis_name="c", subcore_axis_name="s")
    out_type = [
        jax.ShapeDtypeStruct((NC, N, D), jnp.float32),
        jax.ShapeDtypeStruct((NC * N,), jnp.float32),
    ]
    scratch = [
        pltpu.VMEM((N,), jnp.float32),
        pltpu.VMEM((N,), jnp.float32),
        pltpu.VMEM((16,), jnp.float32),
        pltpu.VMEM((16,), jnp.float32),
        pltpu.VMEM((CH,), jnp.int32),
        pltpu.VMEM((CH,), jnp.int32),
        pltpu.VMEM((CH,), jnp.float32),
        pltpu.VMEM((REM,), jnp.int32),
        pltpu.VMEM((REM,), jnp.int32),
        pltpu.VMEM((REM,), jnp.float32),
        pltpu.VMEM((CH,), jnp.float32),
        pltpu.VMEM((CH,), jnp.float32),
        pltpu.VMEM((CH, D), jnp.float32),
        pltpu.VMEM((REM, D), jnp.float32),
        pltpu.VMEM_SHARED((N,), jnp.float32),
        pltpu.VMEM_SHARED((N, D), jnp.float32),
        pltpu.SemaphoreType.DMA,
    ]
    if with_stats:
        out_type += [
            jax.ShapeDtypeStruct((NC * N,), jnp.float32),
            jax.ShapeDtypeStruct((NC * N,), jnp.float32),
        ]
        scratch += [
            pltpu.VMEM_SHARED((N,), jnp.float32),
            pltpu.VMEM_SHARED((N,), jnp.float32),
        ]
    return pl.kernel(
        functools.partial(_edge_body, with_stats),
        mesh=mesh,
        compiler_params=pltpu.CompilerParams(needs_layout_passes=False),
        out_type=out_type,
        scratch_types=scratch,
    )


_edge_kernel1 = _make_edge_kernel(True)
_edge_kernel2 = _make_edge_kernel(False)


def _k1_body(x_ref, wres_ref, bres_ref, w1_ref, asw_ref, adw_ref, alpha_ref,
             we_ref, ae_ref,
             xresa_ref, h1t_ref, as1_ref, ad1_ref, c1_ref, m1_ref):
    x = x_ref[...]
    xr = jnp.dot(x, wres_ref[...], preferred_element_type=jnp.float32)
    xr = xr + bres_ref[...][None, :]
    h = jnp.dot(xr, w1_ref[...], preferred_element_type=jnp.float32)
    al = jnp.sum(alpha_ref[...])
    xresa_ref[...] = al * xr
    h1t_ref[...] = h
    asv = jnp.sum(h * asw_ref[...][None, :], axis=1)
    adv = jnp.sum(h * adw_ref[...][None, :], axis=1)
    as1_ref[...] = asv
    ad1_ref[...] = adv
    c = jnp.sum(we_ref[...] * ae_ref[...])
    m_ub = jnp.max(asv) + jnp.max(adv) + jnp.maximum(c, 0.0)
    m = jnp.maximum(m_ub, 0.2 * m_ub)
    c1_ref[...] = jnp.zeros((16,), jnp.float32) + c
    m1_ref[...] = jnp.zeros((16,), jnp.float32) + m


_k1 = pl.pallas_call(
    _k1_body,
    out_shape=[
        jax.ShapeDtypeStruct((N, D), jnp.float32),
        jax.ShapeDtypeStruct((N, D), jnp.float32),
        jax.ShapeDtypeStruct((N,), jnp.float32),
        jax.ShapeDtypeStruct((N,), jnp.float32),
        jax.ShapeDtypeStruct((16,), jnp.float32),
        jax.ShapeDtypeStruct((16,), jnp.float32),
    ],
)


def _k3_body(accp_ref, denp_ref, cntp_ref, sap_ref, as_ref, ad_ref, h1t_ref,
             c1_ref, m1_ref, b1_ref, w2_ref, asw_ref, adw_ref, we2_ref,
             ae2_ref, h2t_ref, as2_ref, ad2_ref, la_ref, c2_ref, m2_ref):
    cnt = jnp.sum(cntp_ref[...], axis=0)
    sa = jnp.sum(sap_ref[...], axis=0)
    la = sa / jnp.maximum(cnt, 1.0)
    c = jnp.sum(c1_ref[...]) * (1.0 / 16.0)
    m = jnp.sum(m1_ref[...]) * (1.0 / 16.0)
    sl = as_ref[...] + ad_ref[...] + c * la
    sl = jnp.maximum(sl, 0.2 * sl)
    exsl = jnp.exp(sl - m)
    den = jnp.sum(denp_ref[...], axis=0) + exsl
    acc = jnp.sum(accp_ref[...], axis=0) + exsl[:, None] * h1t_ref[...]
    out1 = acc / den[:, None] + b1_ref[...][None, :]
    h1 = jnp.where(out1 > 0, out1, jnp.exp(jnp.minimum(out1, 0.0)) - 1.0)
    h2 = jnp.dot(h1, w2_ref[...], preferred_element_type=jnp.float32)
    h2t_ref[...] = h2
    as2 = jnp.sum(h2 * asw_ref[...][None, :], axis=1)
    ad2 = jnp.sum(h2 * adw_ref[...][None, :], axis=1)
    as2_ref[...] = as2
    ad2_ref[...] = ad2
    la_ref[...] = la
    c2 = jnp.sum(we2_ref[...] * ae2_ref[...])
    m2_ub = jnp.max(as2) + jnp.max(ad2) + jnp.maximum(c2, 0.0)
    m2 = jnp.maximum(m2_ub, 0.2 * m2_ub)
    c2_ref[...] = jnp.zeros((16,), jnp.float32) + c2
    m2_ref[...] = jnp.zeros((16,), jnp.float32) + m2


_k3 = pl.pallas_call(
    _k3_body,
    out_shape=[
        jax.ShapeDtypeStruct((N, D), jnp.float32),
        jax.ShapeDtypeStruct((N,), jnp.float32),
        jax.ShapeDtypeStruct((N,), jnp.float32),
        jax.ShapeDtypeStruct((N,), jnp.float32),
        jax.ShapeDtypeStruct((16,), jnp.float32),
        jax.ShapeDtypeStruct((16,), jnp.float32),
    ],
)


def _k5_body(accp_ref, denp_ref, as_ref, ad_ref, la_ref, h2t_ref, c2_ref,
             m2_ref, b2_ref, xresa_ref, out_ref):
    c = jnp.sum(c2_ref[...]) * (1.0 / 16.0)
    m = jnp.sum(m2_ref[...]) * (1.0 / 16.0)
    sl = as_ref[...] + ad_ref[...] + c * la_ref[...]
    sl = jnp.maximum(sl, 0.2 * sl)
    exsl = jnp.exp(sl - m)
    den = jnp.sum(denp_ref[...], axis=0) + exsl
    acc = jnp.sum(accp_ref[...], axis=0) + exsl[:, None] * h2t_ref[...]
    out_ref[...] = acc / den[:, None] + b2_ref[...][None, :] + xresa_ref[...]


_k5 = pl.pallas_call(
    _k5_body,
    out_shape=jax.ShapeDtypeStruct((N, D), jnp.float32),
)


def kernel(x, edge_index, edge_attr, W_res, b_res, W1, att_src1, att_dst1,
           W_edge1, att_edge1, b1, W2, att_src2, att_dst2, W_edge2,
           att_edge2, b2, alpha):
    src0 = edge_index[0]
    dst0 = edge_index[1]
    ea = edge_attr[:, 0]
    alpha2d = jnp.reshape(alpha, (1, 1))
    we1 = jnp.reshape(W_edge1, (D,))
    we2 = jnp.reshape(W_edge2, (D,))

    xres_a, h1t, as1, ad1, c1, m1 = _k1(x, W_res, b_res, W1, att_src1,
                                        att_dst1, alpha2d, we1, att_edge1)
    acc1, den1, cnt1, sa1 = _edge_kernel1(src0, dst0, ea, as1, ad1, h1t,
                                          c1, m1)
    h2t, as2, ad2, la, c2, m2 = _k3(acc1, den1.reshape(NC, N),
                                    cnt1.reshape(NC, N), sa1.reshape(NC, N),
                                    as1, ad1, h1t, c1, m1, b1, W2, att_src2,
                                    att_dst2, we2, att_edge2)
    acc2, den2 = _edge_kernel2(src0, dst0, ea, as2, ad2, h2t, c2, m2)
    out = _k5(acc2, den2.reshape(NC, N), as2, ad2, la, h2t, c2, m2, b2,
              xres_a)
    return out


# speedup vs baseline: 37.8617x; 1.4193x over previous
import functools

import jax
import jax.numpy as jnp
from jax import lax
from jax.experimental import pallas as pl
from jax.experimental.pallas import tpu as pltpu
from jax.experimental.pallas import tpu_sc as plsc

N = 10000
E = 320000
D = 128

NC = 2
NS = 16
NW = NC * NS
EPT = E // NW
ROWS_AL = 624
ROWS_TL = N - NS * ROWS_AL
CH = 64
NFULL = EPT // CH
REM = EPT - NFULL * CH
G = CH // 16
CG = D // 16


def _scale_rows(rows_ref, exb_ref, nrows):
    def body(r, _):
        e16 = plsc.load_gather(exb_ref, [jnp.zeros((16,), jnp.int32) + r])
        for j in range(CG):
            rows_ref[r, pl.ds(j * 16, 16)] = (
                rows_ref[r, pl.ds(j * 16, 16)] * e16)
        return 0

    lax.fori_loop(0, nrows, body, 0)


def _edge_body(with_stats, *refs):
    if with_stats:
        (src_h, dst_h, ea_h, as_h, ad_h, h_hbm, c_h, m_h,
         acc_p, den_p, cnt_p, sa_p,
         as_l, ad_l, c_l, m_l,
         si0, di0, ei0, si1, di1, ei1,
         srcv16, dstv16, eav16, exb, ones_b, rows0, rows1, rows16,
         den_sh, acc_sh, smi0, smi1, sr0, sr1, sem,
         cnt_sh, sa_sh) = refs
    else:
        (src_h, dst_h, ea_h, as_h, ad_h, h_hbm, c_h, m_h,
         acc_p, den_p,
         as_l, ad_l, c_l, m_l,
         si0, di0, ei0, si1, di1, ei1,
         srcv16, dstv16, eav16, exb, ones_b, rows0, rows1, rows16,
         den_sh, acc_sh, smi0, smi1, sr0, sr1, sem) = refs
        cnt_p = sa_p = cnt_sh = sa_sh = None

    c_ax = lax.axis_index("c")
    s_ax = lax.axis_index("s")
    w = c_ax * NS + s_ax
    base_e = w * EPT

    pltpu.sync_copy(as_h, as_l)
    pltpu.sync_copy(ad_h, ad_l)
    pltpu.sync_copy(c_h, c_l)
    pltpu.sync_copy(m_h, m_l)

    cv = c_l[...]
    mv = m_l[...]

    zv = jnp.zeros((16,), jnp.float32)
    ones16 = jnp.ones((16,), jnp.float32)
    for j in range(G):
        exb[pl.ds(j * 16, 16)] = zv
        ones_b[pl.ds(j * 16, 16)] = ones16

    def zr_body(i, _):
        for j in range(CG):
            rows0[i, pl.ds(j * 16, 16)] = zv
        return 0

    lax.fori_loop(0, CH, zr_body, 0)
    base_r = s_ax * ROWS_AL

    def zfill(off, n):
        for o in range(0, n, CH):
            sz = min(CH, n - o)
            pltpu.sync_copy(rows0.at[pl.ds(0, sz)],
                            acc_sh.at[pl.ds(off + o, sz)])
            pltpu.sync_copy(exb.at[pl.ds(0, sz)],
                            den_sh.at[pl.ds(off + o, sz)])
            if with_stats:
                pltpu.sync_copy(exb.at[pl.ds(0, sz)],
                                cnt_sh.at[pl.ds(off + o, sz)])
                pltpu.sync_copy(exb.at[pl.ds(0, sz)],
                                sa_sh.at[pl.ds(off + o, sz)])

    zfill(base_r, ROWS_AL)

    @pl.when(s_ax == NS - 1)
    def _():
        zfill(NS * ROWS_AL, ROWS_TL)

    plsc.subcore_barrier()

    def issue_idx(si, di, ei, sm, k):
        eb = base_e + k * CH
        pltpu.async_copy(src_h.at[pl.ds(eb, CH)], si, sm)
        pltpu.async_copy(dst_h.at[pl.ds(eb, CH)], di, sm)
        pltpu.async_copy(ea_h.at[pl.ds(eb, CH)], ei, sm)

    def drain_idx(si, di, ei, sm, k):
        eb = base_e + k * CH
        pltpu.make_async_copy(src_h.at[pl.ds(eb, CH)], si, sm).wait()
        pltpu.make_async_copy(dst_h.at[pl.ds(eb, CH)], di, sm).wait()
        pltpu.make_async_copy(ea_h.at[pl.ds(eb, CH)], ei, sm).wait()

    bufs = ((si0, di0, ei0, smi0, rows0, sr0),
            (si1, di1, ei1, smi1, rows1, sr1))

    issue_idx(si0, di0, ei0, smi0, 0)
    drain_idx(si0, di0, ei0, smi0, 0)
    pltpu.async_copy(h_hbm.at[si0], rows0, sr0)
    issue_idx(si1, di1, ei1, smi1, 1)

    def outer(t, _):
        for b in range(2):
            k = 2 * t + b
            si_b, di_b, ei_b, smi_b, rows_b, sr_b = bufs[b]
            si_n, di_n, ei_n, smi_n, rows_n, sr_n = bufs[1 - b]
            for j in range(G):
                sv = si_b[pl.ds(j * 16, 16)]
                dv = di_b[pl.ds(j * 16, 16)]
                eag = ei_b[pl.ds(j * 16, 16)]
                lg = (plsc.load_gather(as_l, [sv])
                      + plsc.load_gather(ad_l, [dv]) + cv * eag)
                lg = jnp.maximum(lg, 0.2 * lg)
                exb[pl.ds(j * 16, 16)] = jnp.exp(lg - mv)
            pltpu.sync_copy(exb, den_sh.at[di_b], add=True)
            if with_stats:
                pltpu.sync_copy(ones_b, cnt_sh.at[di_b], add=True)
                pltpu.sync_copy(ei_b, sa_sh.at[di_b], add=True)

            @pl.when(k + 1 < NFULL)
            def _():
                drain_idx(si_n, di_n, ei_n, smi_n, k + 1)
                pltpu.async_copy(h_hbm.at[si_n], rows_n, sr_n)

            pltpu.make_async_copy(h_hbm.at[si_b], rows_b, sr_b).wait()
            _scale_rows(rows_b, exb, CH)
            pltpu.sync_copy(rows_b, acc_sh.at[di_b], add=True)

            @pl.when(k + 2 < NFULL)
            def _():
                issue_idx(si_b, di_b, ei_b, smi_b, k + 2)
        return 0

    lax.fori_loop(0, NFULL // 2, outer, 0)

    eb = base_e + NFULL * CH
    pltpu.sync_copy(src_h.at[pl.ds(eb, REM)], srcv16)
    pltpu.sync_copy(dst_h.at[pl.ds(eb, REM)], dstv16)
    pltpu.sync_copy(ea_h.at[pl.ds(eb, REM)], eav16)
    cp = pltpu.async_copy(h_hbm.at[srcv16], rows16, sem)
    sv = srcv16[...]
    dv = dstv16[...]
    eag = eav16[...]
    lg = plsc.load_gather(as_l, [sv]) + plsc.load_gather(ad_l, [dv]) + cv * eag
    lg = jnp.maximum(lg, 0.2 * lg)
    exb[pl.ds(0, 16)] = jnp.exp(lg - mv)
    pltpu.sync_copy(exb.at[pl.ds(0, REM)], den_sh.at[dstv16], add=True)
    if with_stats:
        pltpu.sync_copy(ones_b.at[pl.ds(0, REM)], cnt_sh.at[dstv16], add=True)
        pltpu.sync_copy(eav16, sa_sh.at[dstv16], add=True)
    cp.wait()
    _scale_rows(rows16, exb, REM)
    pltpu.sync_copy(rows16, acc_sh.at[dstv16], add=True)

    plsc.subcore_barrier()

    def cp1d(src_sh, dst_h, so, do, n):
        for o in range(0, n, CH):
            sz = min(CH, n - o)
            pltpu.sync_copy(src_sh.at[pl.ds(so + o, sz)],
                            exb.at[pl.ds(0, sz)])
            pltpu.sync_copy(exb.at[pl.ds(0, sz)],
                            dst_h.at[pl.ds(do + o, sz)])

    pltpu.sync_copy(acc_sh.at[pl.ds(base_r, ROWS_AL)],
                    acc_p.at[c_ax, pl.ds(base_r, ROWS_AL)])
    cp1d(den_sh, den_p, base_r, c_ax * N + base_r, ROWS_AL)
    if with_stats:
        cp1d(cnt_sh, cnt_p, base_r, c_ax * N + base_r, ROWS_AL)
        cp1d(sa_sh, sa_p, base_r, c_ax * N + base_r, ROWS_AL)

    @pl.when(s_ax == NS - 1)
    def _():
        pltpu.sync_copy(acc_sh.at[pl.ds(NS * ROWS_AL, ROWS_TL)],
                        acc_p.at[c_ax, pl.ds(NS * ROWS_AL, ROWS_TL)])
        cp1d(den_sh, den_p, NS * ROWS_AL, c_ax * N + NS * ROWS_AL, ROWS_TL)
        if with_stats:
            cp1d(cnt_sh, cnt_p, NS * ROWS_AL, c_ax * N + NS * ROWS_AL,
                 ROWS_TL)
            cp1d(sa_sh, sa_p, NS * ROWS_AL, c_ax * N + NS * ROWS_AL, ROWS_TL)


def _make_edge_kernel(with_stats):
    mesh = plsc.VectorSubcoreMesh(core_axis_name="c", subcore_axis_name="s")
    out_type = [
        jax.ShapeDtypeStruct((NC, N, D), jnp.float32),
        jax.ShapeDtypeStruct((NC * N,), jnp.float32),
    ]
    scratch = [
        pltpu.VMEM((N,), jnp.float32),
        pltpu.VMEM((N,), jnp.float32),
        pltpu.VMEM((16,), jnp.float32),
        pltpu.VMEM((16,), jnp.float32),
        pltpu.VMEM((CH,), jnp.int32),
        pltpu.VMEM((CH,), jnp.int32),
        pltpu.VMEM((CH,), jnp.float32),
        pltpu.VMEM((CH,), jnp.int32),
        pltpu.VMEM((CH,), jnp.int32),
        pltpu.VMEM((CH,), jnp.float32),
        pltpu.VMEM((REM,), jnp.int32),
        pltpu.VMEM((REM,), jnp.int32),
        pltpu.VMEM((REM,), jnp.float32),
        pltpu.VMEM((CH,), jnp.float32),
        pltpu.VMEM((CH,), jnp.float32),
        pltpu.VMEM((CH, D), jnp.float32),
        pltpu.VMEM((CH, D), jnp.float32),
        pltpu.VMEM((REM, D), jnp.float32),
        pltpu.VMEM_SHARED((N,), jnp.float32),
        pltpu.VMEM_SHARED((N, D), jnp.float32),
        pltpu.SemaphoreType.DMA,
        pltpu.SemaphoreType.DMA,
        pltpu.SemaphoreType.DMA,
        pltpu.SemaphoreType.DMA,
        pltpu.SemaphoreType.DMA,
    ]
    if with_stats:
        out_type += [
            jax.ShapeDtypeStruct((NC * N,), jnp.float32),
            jax.ShapeDtypeStruct((NC * N,), jnp.float32),
        ]
        scratch += [
            pltpu.VMEM_SHARED((N,), jnp.float32),
            pltpu.VMEM_SHARED((N,), jnp.float32),
        ]
    return pl.kernel(
        functools.partial(_edge_body, with_stats),
        mesh=mesh,
        compiler_params=pltpu.CompilerParams(needs_layout_passes=False),
        out_type=out_type,
        scratch_types=scratch,
    )


_edge_kernel1 = _make_edge_kernel(True)
_edge_kernel2 = _make_edge_kernel(False)


def _k1_body(x_ref, wres_ref, bres_ref, w1_ref, asw_ref, adw_ref, alpha_ref,
             we_ref, ae_ref,
             xresa_ref, h1t_ref, as1_ref, ad1_ref, c1_ref, m1_ref):
    x = x_ref[...]
    xr = jnp.dot(x, wres_ref[...], preferred_element_type=jnp.float32)
    xr = xr + bres_ref[...][None, :]
    h = jnp.dot(xr, w1_ref[...], preferred_element_type=jnp.float32)
    al = jnp.sum(alpha_ref[...])
    xresa_ref[...] = al * xr
    h1t_ref[...] = h
    asv = jnp.sum(h * asw_ref[...][None, :], axis=1)
    adv = jnp.sum(h * adw_ref[...][None, :], axis=1)
    as1_ref[...] = asv
    ad1_ref[...] = adv
    c = jnp.sum(we_ref[...] * ae_ref[...])
    m_ub = jnp.max(asv) + jnp.max(adv) + jnp.maximum(c, 0.0)
    m = jnp.maximum(m_ub, 0.2 * m_ub)
    c1_ref[...] = jnp.zeros((16,), jnp.float32) + c
    m1_ref[...] = jnp.zeros((16,), jnp.float32) + m


_k1 = pl.pallas_call(
    _k1_body,
    out_shape=[
        jax.ShapeDtypeStruct((N, D), jnp.float32),
        jax.ShapeDtypeStruct((N, D), jnp.float32),
        jax.ShapeDtypeStruct((N,), jnp.float32),
        jax.ShapeDtypeStruct((N,), jnp.float32),
        jax.ShapeDtypeStruct((16,), jnp.float32),
        jax.ShapeDtypeStruct((16,), jnp.float32),
    ],
)


def _k3_body(accp_ref, denp_ref, cntp_ref, sap_ref, as_ref, ad_ref, h1t_ref,
             c1_ref, m1_ref, b1_ref, w2_ref, asw_ref, adw_ref, we2_ref,
             ae2_ref, h2t_ref, as2_ref, ad2_ref, la_ref, c2_ref, m2_ref):
    cnt = jnp.sum(cntp_ref[...], axis=0)
    sa = jnp.sum(sap_ref[...], axis=0)
    la = sa / jnp.maximum(cnt, 1.0)
    c = jnp.sum(c1_ref[...]) * (1.0 / 16.0)
    m = jnp.sum(m1_ref[...]) * (1.0 / 16.0)
    sl = as_ref[...] + ad_ref[...] + c * la
    sl = jnp.maximum(sl, 0.2 * sl)
    exsl = jnp.exp(sl - m)
    den = jnp.sum(denp_ref[...], axis=0) + exsl
    acc = jnp.sum(accp_ref[...], axis=0) + exsl[:, None] * h1t_ref[...]
    out1 = acc / den[:, None] + b1_ref[...][None, :]
    h1 = jnp.where(out1 > 0, out1, jnp.exp(jnp.minimum(out1, 0.0)) - 1.0)
    h2 = jnp.dot(h1, w2_ref[...], preferred_element_type=jnp.float32)
    h2t_ref[...] = h2
    as2 = jnp.sum(h2 * asw_ref[...][None, :], axis=1)
    ad2 = jnp.sum(h2 * adw_ref[...][None, :], axis=1)
    as2_ref[...] = as2
    ad2_ref[...] = ad2
    la_ref[...] = la
    c2 = jnp.sum(we2_ref[...] * ae2_ref[...])
    m2_ub = jnp.max(as2) + jnp.max(ad2) + jnp.maximum(c2, 0.0)
    m2 = jnp.maximum(m2_ub, 0.2 * m2_ub)
    c2_ref[...] = jnp.zeros((16,), jnp.float32) + c2
    m2_ref[...] = jnp.zeros((16,), jnp.float32) + m2


_k3 = pl.pallas_call(
    _k3_body,
    out_shape=[
        jax.ShapeDtypeStruct((N, D), jnp.float32),
        jax.ShapeDtypeStruct((N,), jnp.float32),
        jax.ShapeDtypeStruct((N,), jnp.float32),
        jax.ShapeDtypeStruct((N,), jnp.float32),
        jax.ShapeDtypeStruct((16,), jnp.float32),
        jax.ShapeDtypeStruct((16,), jnp.float32),
    ],
)


def _k5_body(accp_ref, denp_ref, as_ref, ad_ref, la_ref, h2t_ref, c2_ref,
             m2_ref, b2_ref, xresa_ref, out_ref):
    c = jnp.sum(c2_ref[...]) * (1.0 / 16.0)
    m = jnp.sum(m2_ref[...]) * (1.0 / 16.0)
    sl = as_ref[...] + ad_ref[...] + c * la_ref[...]
    sl = jnp.maximum(sl, 0.2 * sl)
    exsl = jnp.exp(sl - m)
    den = jnp.sum(denp_ref[...], axis=0) + exsl
    acc = jnp.sum(accp_ref[...], axis=0) + exsl[:, None] * h2t_ref[...]
    out_ref[...] = acc / den[:, None] + b2_ref[...][None, :] + xresa_ref[...]


_k5 = pl.pallas_call(
    _k5_body,
    out_shape=jax.ShapeDtypeStruct((N, D), jnp.float32),
)


def kernel(x, edge_index, edge_attr, W_res, b_res, W1, att_src1, att_dst1,
           W_edge1, att_edge1, b1, W2, att_src2, att_dst2, W_edge2,
           att_edge2, b2, alpha):
    src0 = edge_index[0]
    dst0 = edge_index[1]
    ea = edge_attr[:, 0]
    alpha2d = jnp.reshape(alpha, (1, 1))
    we1 = jnp.reshape(W_edge1, (D,))
    we2 = jnp.reshape(W_edge2, (D,))

    xres_a, h1t, as1, ad1, c1, m1 = _k1(x, W_res, b_res, W1, att_src1,
                                        att_dst1, alpha2d, we1, att_edge1)
    acc1, den1, cnt1, sa1 = _edge_kernel1(src0, dst0, ea, as1, ad1, h1t,
                                          c1, m1)
    h2t, as2, ad2, la, c2, m2 = _k3(acc1, den1.reshape(NC, N),
                                    cnt1.reshape(NC, N), sa1.reshape(NC, N),
                                    as1, ad1, h1t, c1, m1, b1, W2, att_src2,
                                    att_dst2, we2, att_edge2)
    acc2, den2 = _edge_kernel2(src0, dst0, ea, as2, ad2, h2t, c2, m2)
    out = _k5(acc2, den2.reshape(NC, N), as2, ad2, la, h2t, c2, m2, b2,
              xres_a)
    return out


# speedup vs baseline: 45.8209x; 1.2102x over previous
import functools

import jax
import jax.numpy as jnp
from jax import lax
from jax.experimental import pallas as pl
from jax.experimental.pallas import tpu as pltpu
from jax.experimental.pallas import tpu_sc as plsc

N = 10000
E = 320000
D = 128

NC = 2
NS = 16
NW = NC * NS
EPT = E // NW
ROWS_AL = 624
ROWS_TL = N - NS * ROWS_AL
CH = 64
NFULL = EPT // CH
REM = EPT - NFULL * CH
G = CH // 16
CG = D // 16


def _scale_rows(rows_ref, exb_ref, nrows):
    def body(r, _):
        e16 = plsc.load_gather(exb_ref, [jnp.zeros((16,), jnp.int32) + r])
        for j in range(CG):
            rows_ref[r, pl.ds(j * 16, 16)] = (
                rows_ref[r, pl.ds(j * 16, 16)] * e16)
        return 0

    lax.fori_loop(0, nrows, body, 0)


def _edge_body(with_stats, *refs):
    if with_stats:
        (src_h, dst_h, ea_h, as_h, ad_h, h_hbm, c_h, m_h,
         acc_p, den_p, cnt_p, sa_p,
         as_l, ad_l, c_l, m_l,
         si0, di0, ei0, si1, di1, ei1, dsc0, dsc1,
         srcv16, dstv16, eav16, exb, ones_b, rows0, rows1, rows16,
         den_sh, acc_sh, smi0, smi1, sr0, sr1, sa0, sa1, sem,
         cnt_sh, sa_sh) = refs
    else:
        (src_h, dst_h, ea_h, as_h, ad_h, h_hbm, c_h, m_h,
         acc_p, den_p,
         as_l, ad_l, c_l, m_l,
         si0, di0, ei0, si1, di1, ei1, dsc0, dsc1,
         srcv16, dstv16, eav16, exb, ones_b, rows0, rows1, rows16,
         den_sh, acc_sh, smi0, smi1, sr0, sr1, sa0, sa1, sem) = refs
        cnt_p = sa_p = cnt_sh = sa_sh = None

    c_ax = lax.axis_index("c")
    s_ax = lax.axis_index("s")
    w = c_ax * NS + s_ax
    base_e = w * EPT

    pltpu.sync_copy(as_h, as_l)
    pltpu.sync_copy(ad_h, ad_l)
    pltpu.sync_copy(c_h, c_l)
    pltpu.sync_copy(m_h, m_l)

    cv = c_l[...]
    mv = m_l[...]

    zv = jnp.zeros((16,), jnp.float32)
    ones16 = jnp.ones((16,), jnp.float32)
    for j in range(G):
        exb[pl.ds(j * 16, 16)] = zv
        ones_b[pl.ds(j * 16, 16)] = ones16

    def zr_body(i, _):
        for j in range(CG):
            rows0[i, pl.ds(j * 16, 16)] = zv
        return 0

    lax.fori_loop(0, CH, zr_body, 0)
    base_r = s_ax * ROWS_AL

    def zfill(off, n):
        for o in range(0, n, CH):
            sz = min(CH, n - o)
            pltpu.sync_copy(rows0.at[pl.ds(0, sz)],
                            acc_sh.at[pl.ds(off + o, sz)])
            pltpu.sync_copy(exb.at[pl.ds(0, sz)],
                            den_sh.at[pl.ds(off + o, sz)])
            if with_stats:
                pltpu.sync_copy(exb.at[pl.ds(0, sz)],
                                cnt_sh.at[pl.ds(off + o, sz)])
                pltpu.sync_copy(exb.at[pl.ds(0, sz)],
                                sa_sh.at[pl.ds(off + o, sz)])

    zfill(base_r, ROWS_AL)

    @pl.when(s_ax == NS - 1)
    def _():
        zfill(NS * ROWS_AL, ROWS_TL)

    plsc.subcore_barrier()

    def issue_idx(si, di, ei, sm, k):
        eb = base_e + k * CH
        pltpu.async_copy(src_h.at[pl.ds(eb, CH)], si, sm)
        pltpu.async_copy(dst_h.at[pl.ds(eb, CH)], di, sm)
        pltpu.async_copy(ea_h.at[pl.ds(eb, CH)], ei, sm)

    def drain_idx(si, di, ei, sm, k):
        eb = base_e + k * CH
        pltpu.make_async_copy(src_h.at[pl.ds(eb, CH)], si, sm).wait()
        pltpu.make_async_copy(dst_h.at[pl.ds(eb, CH)], di, sm).wait()
        pltpu.make_async_copy(ea_h.at[pl.ds(eb, CH)], ei, sm).wait()

    bufs = ((si0, di0, ei0, smi0, rows0, sr0, dsc0, sa0),
            (si1, di1, ei1, smi1, rows1, sr1, dsc1, sa1))

    issue_idx(si0, di0, ei0, smi0, 0)
    drain_idx(si0, di0, ei0, smi0, 0)
    pltpu.async_copy(h_hbm.at[si0], rows0, sr0)
    issue_idx(si1, di1, ei1, smi1, 1)

    def outer(t, _):
        for b in range(2):
            k = 2 * t + b
            si_b, di_b, ei_b, smi_b, rows_b, sr_b, dsc_b, sa_b = bufs[b]
            si_n, di_n, ei_n, smi_n, rows_n, sr_n, dsc_n, sa_n = bufs[1 - b]
            for j in range(G):
                sv = si_b[pl.ds(j * 16, 16)]
                dv = di_b[pl.ds(j * 16, 16)]
                eag = ei_b[pl.ds(j * 16, 16)]
                lg = (plsc.load_gather(as_l, [sv])
                      + plsc.load_gather(ad_l, [dv]) + cv * eag)
                lg = jnp.maximum(lg, 0.2 * lg)
                exb[pl.ds(j * 16, 16)] = jnp.exp(lg - mv)
            pltpu.sync_copy(exb, den_sh.at[di_b], add=True)
            if with_stats:
                pltpu.sync_copy(ones_b, cnt_sh.at[di_b], add=True)
                pltpu.sync_copy(ei_b, sa_sh.at[di_b], add=True)

            @pl.when(k > 0)
            def _():
                pltpu.make_async_copy(rows_n, acc_sh.at[dsc_n], sa_n).wait()

            @pl.when(k + 1 < NFULL)
            def _():
                drain_idx(si_n, di_n, ei_n, smi_n, k + 1)
                pltpu.async_copy(h_hbm.at[si_n], rows_n, sr_n)

            pltpu.make_async_copy(h_hbm.at[si_b], rows_b, sr_b).wait()
            _scale_rows(rows_b, exb, CH)
            for j in range(G):
                dsc_b[pl.ds(j * 16, 16)] = di_b[pl.ds(j * 16, 16)]
            pltpu.async_copy(rows_b, acc_sh.at[dsc_b], sa_b, add=True)

            @pl.when(k + 2 < NFULL)
            def _():
                issue_idx(si_b, di_b, ei_b, smi_b, k + 2)
        return 0

    lax.fori_loop(0, NFULL // 2, outer, 0)

    pltpu.make_async_copy(rows1, acc_sh.at[dsc1], sa1).wait()

    eb = base_e + NFULL * CH
    pltpu.sync_copy(src_h.at[pl.ds(eb, REM)], srcv16)
    pltpu.sync_copy(dst_h.at[pl.ds(eb, REM)], dstv16)
    pltpu.sync_copy(ea_h.at[pl.ds(eb, REM)], eav16)
    cp = pltpu.async_copy(h_hbm.at[srcv16], rows16, sem)
    sv = srcv16[...]
    dv = dstv16[...]
    eag = eav16[...]
    lg = plsc.load_gather(as_l, [sv]) + plsc.load_gather(ad_l, [dv]) + cv * eag
    lg = jnp.maximum(lg, 0.2 * lg)
    exb[pl.ds(0, 16)] = jnp.exp(lg - mv)
    pltpu.sync_copy(exb.at[pl.ds(0, REM)], den_sh.at[dstv16], add=True)
    if with_stats:
        pltpu.sync_copy(ones_b.at[pl.ds(0, REM)], cnt_sh.at[dstv16], add=True)
        pltpu.sync_copy(eav16, sa_sh.at[dstv16], add=True)
    cp.wait()
    _scale_rows(rows16, exb, REM)
    pltpu.sync_copy(rows16, acc_sh.at[dstv16], add=True)

    plsc.subcore_barrier()

    def cp1d(src_sh, dst_h, so, do, n):
        for o in range(0, n, CH):
            sz = min(CH, n - o)
            pltpu.sync_copy(src_sh.at[pl.ds(so + o, sz)],
                            exb.at[pl.ds(0, sz)])
            pltpu.sync_copy(exb.at[pl.ds(0, sz)],
                            dst_h.at[pl.ds(do + o, sz)])

    pltpu.sync_copy(acc_sh.at[pl.ds(base_r, ROWS_AL)],
                    acc_p.at[c_ax, pl.ds(base_r, ROWS_AL)])
    cp1d(den_sh, den_p, base_r, c_ax * N + base_r, ROWS_AL)
    if with_stats:
        cp1d(cnt_sh, cnt_p, base_r, c_ax * N + base_r, ROWS_AL)
        cp1d(sa_sh, sa_p, base_r, c_ax * N + base_r, ROWS_AL)

    @pl.when(s_ax == NS - 1)
    def _():
        pltpu.sync_copy(acc_sh.at[pl.ds(NS * ROWS_AL, ROWS_TL)],
                        acc_p.at[c_ax, pl.ds(NS * ROWS_AL, ROWS_TL)])
        cp1d(den_sh, den_p, NS * ROWS_AL, c_ax * N + NS * ROWS_AL, ROWS_TL)
        if with_stats:
            cp1d(cnt_sh, cnt_p, NS * ROWS_AL, c_ax * N + NS * ROWS_AL,
                 ROWS_TL)
            cp1d(sa_sh, sa_p, NS * ROWS_AL, c_ax * N + NS * ROWS_AL, ROWS_TL)


def _make_edge_kernel(with_stats):
    mesh = plsc.VectorSubcoreMesh(core_axis_name="c", subcore_axis_name="s")
    out_type = [
        jax.ShapeDtypeStruct((NC, N, D), jnp.float32),
        jax.ShapeDtypeStruct((NC * N,), jnp.float32),
    ]
    scratch = [
        pltpu.VMEM((N,), jnp.float32),
        pltpu.VMEM((N,), jnp.float32),
        pltpu.VMEM((16,), jnp.float32),
        pltpu.VMEM((16,), jnp.float32),
        pltpu.VMEM((CH,), jnp.int32),
        pltpu.VMEM((CH,), jnp.int32),
        pltpu.VMEM((CH,), jnp.float32),
        pltpu.VMEM((CH,), jnp.int32),
        pltpu.VMEM((CH,), jnp.int32),
        pltpu.VMEM((CH,), jnp.float32),
        pltpu.VMEM((CH,), jnp.int32),
        pltpu.VMEM((CH,), jnp.int32),
        pltpu.VMEM((REM,), jnp.int32),
        pltpu.VMEM((REM,), jnp.int32),
        pltpu.VMEM((REM,), jnp.float32),
        pltpu.VMEM((CH,), jnp.float32),
        pltpu.VMEM((CH,), jnp.float32),
        pltpu.VMEM((CH, D), jnp.float32),
        pltpu.VMEM((CH, D), jnp.float32),
        pltpu.VMEM((REM, D), jnp.float32),
        pltpu.VMEM_SHARED((N,), jnp.float32),
        pltpu.VMEM_SHARED((N, D), jnp.float32),
        pltpu.SemaphoreType.DMA,
        pltpu.SemaphoreType.DMA,
        pltpu.SemaphoreType.DMA,
        pltpu.SemaphoreType.DMA,
        pltpu.SemaphoreType.DMA,
        pltpu.SemaphoreType.DMA,
        pltpu.SemaphoreType.DMA,
    ]
    if with_stats:
        out_type += [
            jax.ShapeDtypeStruct((NC * N,), jnp.float32),
            jax.ShapeDtypeStruct((NC * N,), jnp.float32),
        ]
        scratch += [
            pltpu.VMEM_SHARED((N,), jnp.float32),
            pltpu.VMEM_SHARED((N,), jnp.float32),
        ]
    return pl.kernel(
        functools.partial(_edge_body, with_stats),
        mesh=mesh,
        compiler_params=pltpu.CompilerParams(needs_layout_passes=False),
        out_type=out_type,
        scratch_types=scratch,
    )


_edge_kernel1 = _make_edge_kernel(True)
_edge_kernel2 = _make_edge_kernel(False)


def _k1_body(x_ref, wres_ref, bres_ref, w1_ref, asw_ref, adw_ref, alpha_ref,
             we_ref, ae_ref,
             xresa_ref, h1t_ref, as1_ref, ad1_ref, c1_ref, m1_ref):
    x = x_ref[...]
    xr = jnp.dot(x, wres_ref[...], preferred_element_type=jnp.float32)
    xr = xr + bres_ref[...][None, :]
    h = jnp.dot(xr, w1_ref[...], preferred_element_type=jnp.float32)
    al = jnp.sum(alpha_ref[...])
    xresa_ref[...] = al * xr
    h1t_ref[...] = h
    asv = jnp.sum(h * asw_ref[...][None, :], axis=1)
    adv = jnp.sum(h * adw_ref[...][None, :], axis=1)
    as1_ref[...] = asv
    ad1_ref[...] = adv
    c = jnp.sum(we_ref[...] * ae_ref[...])
    m_ub = jnp.max(asv) + jnp.max(adv) + jnp.maximum(c, 0.0)
    m = jnp.maximum(m_ub, 0.2 * m_ub)
    c1_ref[...] = jnp.zeros((16,), jnp.float32) + c
    m1_ref[...] = jnp.zeros((16,), jnp.float32) + m


_k1 = pl.pallas_call(
    _k1_body,
    out_shape=[
        jax.ShapeDtypeStruct((N, D), jnp.float32),
        jax.ShapeDtypeStruct((N, D), jnp.float32),
        jax.ShapeDtypeStruct((N,), jnp.float32),
        jax.ShapeDtypeStruct((N,), jnp.float32),
        jax.ShapeDtypeStruct((16,), jnp.float32),
        jax.ShapeDtypeStruct((16,), jnp.float32),
    ],
)


def _k3_body(accp_ref, denp_ref, cntp_ref, sap_ref, as_ref, ad_ref, h1t_ref,
             c1_ref, m1_ref, b1_ref, w2_ref, asw_ref, adw_ref, we2_ref,
             ae2_ref, h2t_ref, as2_ref, ad2_ref, la_ref, c2_ref, m2_ref):
    cnt = jnp.sum(cntp_ref[...], axis=0)
    sa = jnp.sum(sap_ref[...], axis=0)
    la = sa / jnp.maximum(cnt, 1.0)
    c = jnp.sum(c1_ref[...]) * (1.0 / 16.0)
    m = jnp.sum(m1_ref[...]) * (1.0 / 16.0)
    sl = as_ref[...] + ad_ref[...] + c * la
    sl = jnp.maximum(sl, 0.2 * sl)
    exsl = jnp.exp(sl - m)
    den = jnp.sum(denp_ref[...], axis=0) + exsl
    acc = jnp.sum(accp_ref[...], axis=0) + exsl[:, None] * h1t_ref[...]
    out1 = acc / den[:, None] + b1_ref[...][None, :]
    h1 = jnp.where(out1 > 0, out1, jnp.exp(jnp.minimum(out1, 0.0)) - 1.0)
    h2 = jnp.dot(h1, w2_ref[...], preferred_element_type=jnp.float32)
    h2t_ref[...] = h2
    as2 = jnp.sum(h2 * asw_ref[...][None, :], axis=1)
    ad2 = jnp.sum(h2 * adw_ref[...][None, :], axis=1)
    as2_ref[...] = as2
    ad2_ref[...] = ad2
    la_ref[...] = la
    c2 = jnp.sum(we2_ref[...] * ae2_ref[...])
    m2_ub = jnp.max(as2) + jnp.max(ad2) + jnp.maximum(c2, 0.0)
    m2 = jnp.maximum(m2_ub, 0.2 * m2_ub)
    c2_ref[...] = jnp.zeros((16,), jnp.float32) + c2
    m2_ref[...] = jnp.zeros((16,), jnp.float32) + m2


_k3 = pl.pallas_call(
    _k3_body,
    out_shape=[
        jax.ShapeDtypeStruct((N, D), jnp.float32),
        jax.ShapeDtypeStruct((N,), jnp.float32),
        jax.ShapeDtypeStruct((N,), jnp.float32),
        jax.ShapeDtypeStruct((N,), jnp.float32),
        jax.ShapeDtypeStruct((16,), jnp.float32),
        jax.ShapeDtypeStruct((16,), jnp.float32),
    ],
)


def _k5_body(accp_ref, denp_ref, as_ref, ad_ref, la_ref, h2t_ref, c2_ref,
             m2_ref, b2_ref, xresa_ref, out_ref):
    c = jnp.sum(c2_ref[...]) * (1.0 / 16.0)
    m = jnp.sum(m2_ref[...]) * (1.0 / 16.0)
    sl = as_ref[...] + ad_ref[...] + c * la_ref[...]
    sl = jnp.maximum(sl, 0.2 * sl)
    exsl = jnp.exp(sl - m)
    den = jnp.sum(denp_ref[...], axis=0) + exsl
    acc = jnp.sum(accp_ref[...], axis=0) + exsl[:, None] * h2t_ref[...]
    out_ref[...] = acc / den[:, None] + b2_ref[...][None, :] + xresa_ref[...]


_k5 = pl.pallas_call(
    _k5_body,
    out_shape=jax.ShapeDtypeStruct((N, D), jnp.float32),
)


def kernel(x, edge_index, edge_attr, W_res, b_res, W1, att_src1, att_dst1,
           W_edge1, att_edge1, b1, W2, att_src2, att_dst2, W_edge2,
           att_edge2, b2, alpha):
    src0 = edge_index[0]
    dst0 = edge_index[1]
    ea = edge_attr[:, 0]
    alpha2d = jnp.reshape(alpha, (1, 1))
    we1 = jnp.reshape(W_edge1, (D,))
    we2 = jnp.reshape(W_edge2, (D,))

    xres_a, h1t, as1, ad1, c1, m1 = _k1(x, W_res, b_res, W1, att_src1,
                                        att_dst1, alpha2d, we1, att_edge1)
    acc1, den1, cnt1, sa1 = _edge_kernel1(src0, dst0, ea, as1, ad1, h1t,
                                          c1, m1)
    h2t, as2, ad2, la, c2, m2 = _k3(acc1, den1.reshape(NC, N),
                                    cnt1.reshape(NC, N), sa1.reshape(NC, N),
                                    as1, ad1, h1t, c1, m1, b1, W2, att_src2,
                                    att_dst2, we2, att_edge2)
    acc2, den2 = _edge_kernel2(src0, dst0, ea, as2, ad2, h2t, c2, m2)
    out = _k5(acc2, den2.reshape(NC, N), as2, ad2, la, h2t, c2, m2, b2,
              xres_a)
    return out
